# baseline (speedup 1.0000x reference)
import functools

import jax
import jax.numpy as jnp
from jax.experimental import pallas as pl
from jax.experimental.pallas import tpu as pltpu

_NUM_EXPERTS = 8
_RANK = 16
_SCALING = 2.0
_LORA_COLS = _NUM_EXPERTS * _RANK


def _fused_kernel(x_ref, w_ref, a_ref, b_ref, r_ref, o_ref):
    x = x_ref[...]
    xb = x.astype(jnp.bfloat16)

    logits = jax.lax.dot_general(
        x, r_ref[...], (((1,), (1,)), ((), ())),
        preferred_element_type=jnp.float32)
    w = jax.nn.softmax(logits, axis=-1)
    lane = jax.lax.broadcasted_iota(jnp.int32, w.shape, 1)
    m1 = jnp.max(w, axis=-1, keepdims=True)
    i1 = jnp.min(jnp.where(w == m1, lane, _NUM_EXPERTS), axis=-1, keepdims=True)
    w2 = jnp.where(lane == i1, -1.0, w)
    m2 = jnp.max(w2, axis=-1, keepdims=True)
    i2 = jnp.min(jnp.where(w2 == m2, lane, _NUM_EXPERTS), axis=-1, keepdims=True)
    denom = m1 + m2
    c1 = (m1 / denom) * _SCALING
    c2 = (m2 / denom) * _SCALING

    egrp = jax.lax.broadcasted_iota(jnp.int32, (x.shape[0], _LORA_COLS), 1) // _RANK
    scale = jnp.where(egrp == i1, c1, 0.0) + jnp.where(egrp == i2, c2, 0.0)

    r = jax.lax.dot_general(
        xb, a_ref[...], (((1,), (1,)), ((), ())),
        preferred_element_type=jnp.float32)
    rs = (r * scale).astype(jnp.bfloat16)
    lora = jax.lax.dot_general(
        rs, b_ref[...], (((1,), (0,)), ((), ())),
        preferred_element_type=jnp.float32)

    base = jax.lax.dot_general(
        xb, w_ref[...], (((1,), (1,)), ((), ())),
        preferred_element_type=jnp.float32)
    o_ref[...] = base + lora


@functools.partial(jax.jit, static_argnames=("interpret",))
def kernel(x, weight, lora_A, lora_B, router_w, interpret=False):
    B, T, D = x.shape
    out_f = weight.shape[0]
    x2 = x.reshape(B * T, D)
    w_bf = weight.astype(jnp.bfloat16)
    a_all = lora_A.reshape(_LORA_COLS, D).astype(jnp.bfloat16)
    b_all = lora_B.transpose(0, 2, 1).reshape(_LORA_COLS, out_f).astype(jnp.bfloat16)

    tm = 512
    grid = (B * T // tm,)
    out = pl.pallas_call(
        _fused_kernel,
        grid=grid,
        in_specs=[
            pl.BlockSpec((tm, D), lambda i: (i, 0)),
            pl.BlockSpec((out_f, D), lambda i: (0, 0)),
            pl.BlockSpec((_LORA_COLS, D), lambda i: (0, 0)),
            pl.BlockSpec((_LORA_COLS, out_f), lambda i: (0, 0)),
            pl.BlockSpec((_NUM_EXPERTS, D), lambda i: (0, 0)),
        ],
        out_specs=pl.BlockSpec((tm, out_f), lambda i: (i, 0)),
        out_shape=jax.ShapeDtypeStruct((B * T, out_f), jnp.float32),
        compiler_params=pltpu.CompilerParams(
            dimension_semantics=("parallel",)),
        interpret=interpret,
    )(x2, w_bf, a_all, b_all, router_w)
    return out.reshape(B, T, out_f)


# speedup vs baseline: 1.2937x; 1.2937x over previous
import functools

import jax
import jax.numpy as jnp
from jax.experimental import pallas as pl
from jax.experimental.pallas import tpu as pltpu

_NUM_EXPERTS = 8
_RANK = 16
_SCALING = 2.0
_LORA_COLS = _NUM_EXPERTS * _RANK


def _fused_kernel(x_ref, w_ref, a_ref, b_ref, r_ref, o_ref, wa_ref):
    d = x_ref.shape[1]
    out_f = w_ref.shape[0]

    @pl.when(pl.program_id(0) == 0)
    def _cast_weights():
        wa_ref[:out_f, :] = w_ref[...].astype(jnp.bfloat16)
        wa_ref[out_f:, :] = a_ref[...].astype(jnp.bfloat16)

    x = x_ref[...]
    xb = x.astype(jnp.bfloat16)

    logits = jax.lax.dot_general(
        x, r_ref[...], (((1,), (1,)), ((), ())),
        preferred_element_type=jnp.float32)
    lane = jax.lax.broadcasted_iota(jnp.int32, logits.shape, 1)
    m1 = jnp.max(logits, axis=-1, keepdims=True)
    i1 = jnp.min(jnp.where(logits == m1, lane, _NUM_EXPERTS),
                 axis=-1, keepdims=True)
    l2 = jnp.where(lane == i1, -1e30, logits)
    m2 = jnp.max(l2, axis=-1, keepdims=True)
    i2 = jnp.min(jnp.where(l2 == m2, lane, _NUM_EXPERTS),
                 axis=-1, keepdims=True)
    e = jnp.exp(m2 - m1)
    inv = _SCALING / (1.0 + e)
    c1 = inv
    c2 = e * inv

    egrp = jax.lax.broadcasted_iota(
        jnp.int32, (x.shape[0], _LORA_COLS), 1) // _RANK
    scale = jnp.where(egrp == i1, c1, 0.0) + jnp.where(egrp == i2, c2, 0.0)

    y = jax.lax.dot_general(
        xb, wa_ref[...], (((1,), (1,)), ((), ())),
        preferred_element_type=jnp.float32)
    rs = (y[:, out_f:] * scale).astype(jnp.bfloat16)
    lora = jax.lax.dot_general(
        rs, b_ref[...], (((1,), (0,)), ((), ())),
        preferred_element_type=jnp.float32)
    o_ref[...] = y[:, :out_f] + lora


@functools.partial(jax.jit, static_argnames=("interpret",))
def kernel(x, weight, lora_A, lora_B, router_w, interpret=False):
    B, T, D = x.shape
    out_f = weight.shape[0]
    x2 = x.reshape(B * T, D)
    a_all = lora_A.reshape(_LORA_COLS, D)
    b_all = lora_B.transpose(0, 2, 1).reshape(_LORA_COLS, out_f).astype(
        jnp.bfloat16)

    tm = 512
    grid = (B * T // tm,)
    out = pl.pallas_call(
        _fused_kernel,
        grid=grid,
        in_specs=[
            pl.BlockSpec((tm, D), lambda i: (i, 0)),
            pl.BlockSpec((out_f, D), lambda i: (0, 0)),
            pl.BlockSpec((_LORA_COLS, D), lambda i: (0, 0)),
            pl.BlockSpec((_LORA_COLS, out_f), lambda i: (0, 0)),
            pl.BlockSpec((_NUM_EXPERTS, D), lambda i: (0, 0)),
        ],
        out_specs=pl.BlockSpec((tm, out_f), lambda i: (i, 0)),
        out_shape=jax.ShapeDtypeStruct((B * T, out_f), jnp.float32),
        scratch_shapes=[pltpu.VMEM((out_f + _LORA_COLS, D), jnp.bfloat16)],
        interpret=interpret,
    )(x2, weight, a_all, b_all, router_w)
    return out.reshape(B, T, out_f)
